# baseline (speedup 1.0000x reference)
import jax
import jax.numpy as jnp
from jax import lax
from jax.experimental import pallas as pl
from jax.experimental.pallas import tpu as pltpu
from jax.experimental.pallas import tpu_sc as plsc

N_NODES = 10000
N_EDGES = 320000
D_FEAT = 128

NC = 2
NS = 16
LANES = 16
NROW = 3
NIDX = 4

CHUNK = 96
NCHUNK = 105
EPT = CHUNK * NCHUNK
NGCHUNK = NCHUNK * NC * NS
EPT_RAW = N_EDGES // (NC * NS)
RPT = 624
REM = N_NODES - RPT * NS

_DNUMS = lax.GatherDimensionNumbers(
    offset_dims=(), collapsed_slice_dims=(0,), start_index_map=(0,))


def _scale(rows_v, val_v, n):
  def scale_group(g, _):
    val16 = val_v[pl.ds(g * LANES, LANES)]
    for i in range(LANES):
      vsplat = lax.gather(
          val16, jnp.full((LANES, 1), i, jnp.int32), _DNUMS,
          slice_sizes=(1,), mode=lax.GatherScatterMode.PROMISE_IN_BOUNDS)
      e = g * LANES + i
      for j in range(D_FEAT // LANES):
        sl = (e, pl.ds(j * LANES, LANES))
        rows_v[sl] = rows_v[sl] * vsplat
    return 0
  lax.fori_loop(0, n // LANES, scale_group, 0)


def _spmm_body(h_hbm, row_hbm, col_hbm, val_hbm, p_hbm, *refs):
  rows = refs[0:3]
  rowi = refs[3:7]
  coli = refs[7:11]
  vbuf = refs[11:15]
  acc_sh = refs[15]
  gat_sem = refs[16:19]
  scat_sem = refs[19:22]
  idx_sem = refs[22:26]

  c = lax.axis_index("c")
  s = lax.axis_index("s")
  tile = c * NS + s
  ebase = tile * EPT

  def idx_start(k, sl):
    base = ebase + k * CHUNK
    pltpu.async_copy(row_hbm.at[pl.ds(base, CHUNK)], rowi[sl], idx_sem[sl])
    pltpu.async_copy(col_hbm.at[pl.ds(base, CHUNK)], coli[sl], idx_sem[sl])
    pltpu.async_copy(val_hbm.at[pl.ds(base, CHUNK)], vbuf[sl], idx_sem[sl])

  def idx_wait(k, sl):
    base = ebase + k * CHUNK
    pltpu.make_async_copy(
        row_hbm.at[pl.ds(base, CHUNK)], rowi[sl], idx_sem[sl]).wait()
    pltpu.make_async_copy(
        col_hbm.at[pl.ds(base, CHUNK)], coli[sl], idx_sem[sl]).wait()
    pltpu.make_async_copy(
        val_hbm.at[pl.ds(base, CHUNK)], vbuf[sl], idx_sem[sl]).wait()

  def scat_wait(rsl, isl):
    pltpu.make_async_copy(rows[rsl], acc_sh.at[rowi[isl]],
                          scat_sem[rsl]).wait()

  idx_start(0, 0)
  idx_wait(0, 0)
  pltpu.async_copy(h_hbm.at[coli[0]], rows[0], gat_sem[0])
  idx_start(1, 1)

  def zero_body(r, _):
    for j in range(D_FEAT // LANES):
      rows[1][r, pl.ds(j * LANES, LANES)] = jnp.zeros((LANES,), jnp.float32)
    return 0
  lax.fori_loop(0, CHUNK, zero_body, 0)

  def zero_rows(start, cnt):
    done = 0
    while done < cnt:
      step = min(CHUNK, cnt - done)
      pltpu.sync_copy(rows[1].at[pl.ds(0, step)],
                      acc_sh.at[pl.ds(start + done, step)])
      done += step

  zero_rows(s * RPT, RPT)

  @pl.when(s == NS - 1)
  def _():
    zero_rows(NS * RPT, REM)

  plsc.subcore_barrier()

  def iteration(k, rsl, isl):
    rsl1 = (rsl + 1) % NROW
    isl1 = (isl + 1) % NIDX
    isl2 = (isl + 2) % NIDX

    @pl.when(jnp.logical_and(k >= 2, k + 1 < NCHUNK))
    def _():
      scat_wait(rsl1, isl2)

    @pl.when(k + 1 < NCHUNK)
    def _():
      idx_wait(k + 1, isl1)
      half = CHUNK // 2
      pltpu.async_copy(h_hbm.at[coli[isl1].at[pl.ds(0, half)]],
                       rows[rsl1].at[pl.ds(0, half)], gat_sem[rsl1])
      pltpu.async_copy(h_hbm.at[coli[isl1].at[pl.ds(half, half)]],
                       rows[rsl1].at[pl.ds(half, half)], gat_sem[rsl1])

    @pl.when(k + 2 < NCHUNK)
    def _():
      idx_start(k + 2, isl2)

    pltpu.make_async_copy(h_hbm.at[coli[isl]], rows[rsl],
                          gat_sem[rsl]).wait()
    _scale(rows[rsl], vbuf[isl], CHUNK)
    pltpu.async_copy(rows[rsl], acc_sh.at[rowi[isl]], scat_sem[rsl],
                     add=True)

  def twelve(t, _):
    k = t * 12
    for i in range(12):
      iteration(k + i, i % NROW, i % NIDX)
    return 0
  lax.fori_loop(0, NCHUNK // 12, twelve, 0)
  for k in range(NCHUNK - NCHUNK % 12, NCHUNK):
    iteration(k, k % NROW, k % NIDX)

  for k in range(NCHUNK - 3, NCHUNK):
    scat_wait(k % NROW, k % NIDX)

  plsc.subcore_barrier()
  pltpu.sync_copy(acc_sh.at[pl.ds(s * RPT, RPT)],
                  p_hbm.at[c, pl.ds(s * RPT, RPT)])

  @pl.when(s == NS - 1)
  def _():
    pltpu.sync_copy(acc_sh.at[pl.ds(NS * RPT, REM)],
                    p_hbm.at[c, pl.ds(NS * RPT, REM)])


@jax.jit
def _spmm(h, row, col, val):
  mesh = plsc.VectorSubcoreMesh(core_axis_name="c", subcore_axis_name="s")
  scratch = (
      [pltpu.VMEM((CHUNK, D_FEAT), jnp.float32)] * NROW
      + [pltpu.VMEM((CHUNK,), jnp.int32)] * NIDX
      + [pltpu.VMEM((CHUNK,), jnp.int32)] * NIDX
      + [pltpu.VMEM((CHUNK,), jnp.float32)] * NIDX
      + [pltpu.VMEM_SHARED((N_NODES, D_FEAT), jnp.float32)]
      + [pltpu.SemaphoreType.DMA] * (2 * NROW + NIDX)
  )
  return pl.kernel(
      _spmm_body,
      out_type=jax.ShapeDtypeStruct((NC, N_NODES, D_FEAT), jnp.float32),
      mesh=mesh,
      scratch_types=scratch,
      name="gcn_spmm_sc",
  )(h, row, col, val)


def _dense_body(p_ref, w_ref, b_ref, o_ref):
  x = p_ref[0] + p_ref[1]
  y = jnp.dot(x, w_ref[...], preferred_element_type=jnp.float32) + b_ref[...]
  o_ref[...] = jnp.maximum(y, 0.0)


BLK = 1000


@jax.jit
def _dense(p, w, b):
  b2 = b.reshape(1, D_FEAT)
  grid = (N_NODES // BLK,)
  return pl.pallas_call(
      _dense_body,
      grid=grid,
      in_specs=[
          pl.BlockSpec((NC, BLK, D_FEAT), lambda i: (0, i, 0)),
          pl.BlockSpec((D_FEAT, D_FEAT), lambda i: (0, 0)),
          pl.BlockSpec((1, D_FEAT), lambda i: (0, 0)),
      ],
      out_specs=pl.BlockSpec((BLK, D_FEAT), lambda i: (i, 0)),
      out_shape=jax.ShapeDtypeStruct((N_NODES, D_FEAT), jnp.float32),
      name="gcn_dense_tc",
  )(p, w, b2)


def kernel(H, edge_index, edge_values, W0, b0, W1, b1):
  npad = EPT - EPT_RAW
  spread = jnp.broadcast_to((jnp.arange(npad, dtype=jnp.int32) * 101) % N_NODES,
                            (NC * NS, npad))
  row = jnp.concatenate(
      [edge_index[0].astype(jnp.int32).reshape(NC * NS, EPT_RAW), spread],
      axis=1).reshape(-1)
  col = jnp.concatenate(
      [edge_index[1].astype(jnp.int32).reshape(NC * NS, EPT_RAW), spread],
      axis=1).reshape(-1)
  val = jnp.pad(edge_values.astype(jnp.float32).reshape(NC * NS, EPT_RAW),
                ((0, 0), (0, npad))).reshape(-1)
  p0 = _spmm(H, row, col, val)
  h1 = _dense(p0, W0, b0)
  p1 = _spmm(h1, row, col, val)
  h2 = _dense(p1, W1, b1)
  return h2


# speedup vs baseline: 1.0635x; 1.0635x over previous
import jax
import jax.numpy as jnp
from jax import lax
from jax.experimental import pallas as pl
from jax.experimental.pallas import tpu as pltpu
from jax.experimental.pallas import tpu_sc as plsc

N_NODES = 10000
N_EDGES = 320000
D_FEAT = 128

NC = 2
NS = 16
LANES = 16
NROW = 4
NIDX = 5

CHUNK = 80
NCHUNK = 125
EPT = CHUNK * NCHUNK
EPT_RAW = N_EDGES // (NC * NS)
RPT = 624
REM = N_NODES - RPT * NS

_DNUMS = lax.GatherDimensionNumbers(
    offset_dims=(), collapsed_slice_dims=(0,), start_index_map=(0,))


def _scale(rows_v, val_v, n):
  def scale_group(g, _):
    val16 = val_v[pl.ds(g * LANES, LANES)]
    for i in range(LANES):
      vsplat = lax.gather(
          val16, jnp.full((LANES, 1), i, jnp.int32), _DNUMS,
          slice_sizes=(1,), mode=lax.GatherScatterMode.PROMISE_IN_BOUNDS)
      e = g * LANES + i
      for j in range(D_FEAT // LANES):
        sl = (e, pl.ds(j * LANES, LANES))
        rows_v[sl] = rows_v[sl] * vsplat
    return 0
  lax.fori_loop(0, n // LANES, scale_group, 0)


def _spmm_body(h_hbm, row_hbm, col_hbm, val_hbm, p_hbm, *refs):
  rows = refs[0:NROW]
  o = NROW
  rowi = refs[o:o + NIDX]
  o += NIDX
  coli = refs[o:o + NIDX]
  o += NIDX
  vbuf = refs[o:o + NIDX]
  o += NIDX
  acc_sh = refs[o]
  o += 1
  gat_sem = refs[o:o + NROW]
  o += NROW
  scat_sem = refs[o:o + NROW]
  o += NROW
  idx_sem = refs[o:o + NIDX]

  c = lax.axis_index("c")
  s = lax.axis_index("s")
  tile = c * NS + s
  ebase = tile * EPT

  def idx_start(k, sl):
    base = ebase + k * CHUNK
    pltpu.async_copy(row_hbm.at[pl.ds(base, CHUNK)], rowi[sl], idx_sem[sl])
    pltpu.async_copy(col_hbm.at[pl.ds(base, CHUNK)], coli[sl], idx_sem[sl])
    pltpu.async_copy(val_hbm.at[pl.ds(base, CHUNK)], vbuf[sl], idx_sem[sl])

  def idx_wait(k, sl):
    base = ebase + k * CHUNK
    pltpu.make_async_copy(
        row_hbm.at[pl.ds(base, CHUNK)], rowi[sl], idx_sem[sl]).wait()
    pltpu.make_async_copy(
        col_hbm.at[pl.ds(base, CHUNK)], coli[sl], idx_sem[sl]).wait()
    pltpu.make_async_copy(
        val_hbm.at[pl.ds(base, CHUNK)], vbuf[sl], idx_sem[sl]).wait()

  def scat_wait(rsl, isl):
    pltpu.make_async_copy(rows[rsl], acc_sh.at[rowi[isl]],
                          scat_sem[rsl]).wait()

  def gat_start(k, isl, rsl):
    pltpu.async_copy(h_hbm.at[coli[isl]], rows[rsl], gat_sem[rsl])

  idx_start(0, 0)
  idx_start(1, 1)
  idx_start(2, 2)
  idx_wait(0, 0)
  gat_start(0, 0, 0)
  idx_wait(1, 1)
  gat_start(1, 1, 1)

  def zero_body(r, _):
    for j in range(D_FEAT // LANES):
      rows[3][r, pl.ds(j * LANES, LANES)] = jnp.zeros((LANES,), jnp.float32)
    return 0
  lax.fori_loop(0, CHUNK, zero_body, 0)

  def zero_rows(start, cnt):
    done = 0
    while done < cnt:
      step = min(CHUNK, cnt - done)
      pltpu.sync_copy(rows[3].at[pl.ds(0, step)],
                      acc_sh.at[pl.ds(start + done, step)])
      done += step

  zero_rows(s * RPT, RPT)

  @pl.when(s == NS - 1)
  def _():
    zero_rows(NS * RPT, REM)

  plsc.subcore_barrier()

  def iteration(k, rsl, isl):
    rsl2 = (rsl + 2) % NROW
    isl2 = (isl + 2) % NIDX
    isl3 = (isl + 3) % NIDX

    @pl.when(jnp.logical_and(k >= 2, k + 2 < NCHUNK))
    def _():
      scat_wait(rsl2, isl3)

    @pl.when(k + 2 < NCHUNK)
    def _():
      idx_wait(k + 2, isl2)
      gat_start(k + 2, isl2, rsl2)

    @pl.when(k + 3 < NCHUNK)
    def _():
      idx_start(k + 3, isl3)

    pltpu.make_async_copy(h_hbm.at[coli[isl]], rows[rsl],
                          gat_sem[rsl]).wait()
    _scale(rows[rsl], vbuf[isl], CHUNK)
    pltpu.async_copy(rows[rsl], acc_sh.at[rowi[isl]], scat_sem[rsl],
                     add=True)

  period = 20
  def period_body(t, _):
    k = t * period
    for i in range(period):
      iteration(k + i, i % NROW, i % NIDX)
    return 0
  lax.fori_loop(0, NCHUNK // period, period_body, 0)
  for k in range(NCHUNK - NCHUNK % period, NCHUNK):
    iteration(k, k % NROW, k % NIDX)

  for k in range(NCHUNK - 4, NCHUNK):
    scat_wait(k % NROW, k % NIDX)

  plsc.subcore_barrier()
  pltpu.sync_copy(acc_sh.at[pl.ds(s * RPT, RPT)],
                  p_hbm.at[c, pl.ds(s * RPT, RPT)])

  @pl.when(s == NS - 1)
  def _():
    pltpu.sync_copy(acc_sh.at[pl.ds(NS * RPT, REM)],
                    p_hbm.at[c, pl.ds(NS * RPT, REM)])


@jax.jit
def _spmm(h, row, col, val):
  mesh = plsc.VectorSubcoreMesh(core_axis_name="c", subcore_axis_name="s")
  scratch = (
      [pltpu.VMEM((CHUNK, D_FEAT), jnp.float32)] * NROW
      + [pltpu.VMEM((CHUNK,), jnp.int32)] * NIDX
      + [pltpu.VMEM((CHUNK,), jnp.int32)] * NIDX
      + [pltpu.VMEM((CHUNK,), jnp.float32)] * NIDX
      + [pltpu.VMEM_SHARED((N_NODES, D_FEAT), jnp.float32)]
      + [pltpu.SemaphoreType.DMA] * (2 * NROW + NIDX)
  )
  return pl.kernel(
      _spmm_body,
      out_type=jax.ShapeDtypeStruct((NC, N_NODES, D_FEAT), jnp.float32),
      mesh=mesh,
      scratch_types=scratch,
      name="gcn_spmm_sc",
  )(h, row, col, val)


def _dense_body(p_ref, w_ref, b_ref, o_ref):
  x = p_ref[0] + p_ref[1]
  y = jnp.dot(x, w_ref[...], preferred_element_type=jnp.float32) + b_ref[...]
  o_ref[...] = jnp.maximum(y, 0.0)


BLK = 1000


@jax.jit
def _dense(p, w, b):
  b2 = b.reshape(1, D_FEAT)
  grid = (N_NODES // BLK,)
  return pl.pallas_call(
      _dense_body,
      grid=grid,
      in_specs=[
          pl.BlockSpec((NC, BLK, D_FEAT), lambda i: (0, i, 0)),
          pl.BlockSpec((D_FEAT, D_FEAT), lambda i: (0, 0)),
          pl.BlockSpec((1, D_FEAT), lambda i: (0, 0)),
      ],
      out_specs=pl.BlockSpec((BLK, D_FEAT), lambda i: (i, 0)),
      out_shape=jax.ShapeDtypeStruct((N_NODES, D_FEAT), jnp.float32),
      name="gcn_dense_tc",
  )(p, w, b2)


def kernel(H, edge_index, edge_values, W0, b0, W1, b1):
  npad = EPT - EPT_RAW
  spread = jnp.broadcast_to((jnp.arange(npad, dtype=jnp.int32) * 101) % N_NODES,
                            (NC * NS, npad))
  row = jnp.concatenate(
      [edge_index[0].astype(jnp.int32).reshape(NC * NS, EPT_RAW), spread],
      axis=1).reshape(-1)
  col = jnp.concatenate(
      [edge_index[1].astype(jnp.int32).reshape(NC * NS, EPT_RAW), spread],
      axis=1).reshape(-1)
  val = jnp.pad(edge_values.astype(jnp.float32).reshape(NC * NS, EPT_RAW),
                ((0, 0), (0, npad))).reshape(-1)
  p0 = _spmm(H, row, col, val)
  h1 = _dense(p0, W0, b0)
  p1 = _spmm(h1, row, col, val)
  h2 = _dense(p1, W1, b1)
  return h2
